# baseline (speedup 1.0000x reference)
import functools

import jax
import jax.numpy as jnp
from jax import lax
from jax.experimental import pallas as pl
from jax.experimental.pallas import tpu as pltpu

_LANES = 128


def _iota2(shape, offset=0):
    r = lax.broadcasted_iota(jnp.int32, shape, 0)
    c = lax.broadcasted_iota(jnp.int32, shape, 1)
    return r * _LANES + c + offset


def _partner(a, bit, t_row_shift, t_lane_shift, axis):
    n = a.shape[axis]
    if axis == 0:
        sh = t_row_shift
    else:
        sh = t_lane_shift
    from_plus = pltpu.roll(a, n - sh, axis)
    from_minus = pltpu.roll(a, sh, axis)
    return jnp.where(bit, from_minus, from_plus)


def _ce_vals(k, v, pk, pv, take_small, tie):
    if tie:
        less = (k < pk) | ((k == pk) & (v < pv))
    else:
        less = k < pk
    keep = less == take_small
    nk = jnp.where(keep, k, pk)
    if v is None:
        return nk, None
    return nk, jnp.where(keep, v, pv)


def _local_passes(k, v, gi, s, tie, lgc):
    desc = ((gi >> s) & 1) == 1
    t_start = jnp.minimum(jnp.int32(s) - 1, lgc - 1)
    has_v = v is not None

    def row_body(i, kv):
        k, v = kv if has_v else (kv, None)
        t = t_start - i
        d = jnp.left_shift(jnp.int32(1), t)
        dr = jnp.left_shift(jnp.int32(1), t - 7)
        bit = (gi & d) != 0
        take_small = bit == desc
        pk = _partner(k, bit, dr, None, 0)
        pv = _partner(v, bit, dr, None, 0) if has_v else None
        nk, nv = _ce_vals(k, v, pk, pv, take_small, tie)
        return (nk, nv) if has_v else nk

    n_row = jnp.maximum(t_start - 6, 0)
    carry = (k, v) if has_v else k
    carry = lax.fori_loop(0, n_row, row_body, carry)

    def lane_body(i, kv):
        k, v = kv if has_v else (kv, None)
        t = jnp.minimum(t_start, 6) - i
        d = jnp.left_shift(jnp.int32(1), t)
        bit = (gi & d) != 0
        take_small = bit == desc
        pk = _partner(k, bit, None, d, 1)
        pv = _partner(v, bit, None, d, 1) if has_v else None
        nk, nv = _ce_vals(k, v, pk, pv, take_small, tie)
        return (nk, nv) if has_v else nk

    n_lane = jnp.minimum(t_start + 1, 7)
    carry = lax.fori_loop(0, n_lane, lane_body, carry)
    return carry if has_v else (carry, None)


def _sort_refs(k_ref, v_ref, tie, pad_rows, chunk_rows, lgn, lgc):
    nch = pad_rows // chunk_rows
    ce = chunk_rows * _LANES
    li = _iota2((chunk_rows, _LANES))
    has_v = v_ref is not None

    def chunk_body(c, _):
        sl = pl.ds(c * chunk_rows, chunk_rows)
        gi = li + c * ce
        k = k_ref[sl, :]
        v = v_ref[sl, :] if has_v else None

        def s_body(s, kv):
            kk, vv = kv if has_v else (kv, None)
            kk, vv = _local_passes(kk, vv, gi, s, tie, lgc)
            return (kk, vv) if has_v else kk

        carry = (k, v) if has_v else k
        carry = lax.fori_loop(1, lgc + 1, s_body, carry)
        k, v = carry if has_v else (carry, None)
        k_ref[sl, :] = k
        if has_v:
            v_ref[sl, :] = v
        return 0

    lax.fori_loop(0, nch, chunk_body, 0)

    for s in range(lgc + 1, lgn + 1):
        for t in range(s - 1, lgc - 1, -1):
            dch = 1 << (t - lgc)
            for c in range(nch):
                if c & dch:
                    continue
                p = c + dch
                desc = ((c >> (s - lgc)) & 1) == 1
                sa = slice(c * chunk_rows, (c + 1) * chunk_rows)
                sb = slice(p * chunk_rows, (p + 1) * chunk_rows)
                ka = k_ref[sa, :]
                kb = k_ref[sb, :]
                if tie:
                    va = v_ref[sa, :]
                    vb = v_ref[sb, :]
                    less = (ka < kb) | ((ka == kb) & (va < vb))
                elif has_v:
                    va = v_ref[sa, :]
                    vb = v_ref[sb, :]
                    less = ka < kb
                else:
                    less = ka < kb
                if desc:
                    less = ~less
                k_ref[sa, :] = jnp.where(less, ka, kb)
                k_ref[sb, :] = jnp.where(less, kb, ka)
                if has_v:
                    v_ref[sa, :] = jnp.where(less, va, vb)
                    v_ref[sb, :] = jnp.where(less, vb, va)

        def chunk_body2(c, _, s=s):
            sl = pl.ds(c * chunk_rows, chunk_rows)
            gi = li + c * ce
            k = k_ref[sl, :]
            v = v_ref[sl, :] if has_v else None
            k, v = _local_passes(k, v, gi, s, tie, lgc)
            k_ref[sl, :] = k
            if has_v:
                v_ref[sl, :] = v
            return 0

        lax.fori_loop(0, nch, chunk_body2, 0)


def _efdm_row_kernel(x_ref, y_ref, o_ref, kx, ix, ky, *, rows, pad_rows,
                     chunk_rows, lgn, lgc):
    pad = pad_rows - rows
    kx[0:rows, :] = x_ref[0]
    ix[...] = _iota2((pad_rows, _LANES))
    ky[0:rows, :] = y_ref[0]
    if pad:
        inf_pad = jnp.full((pad, _LANES), jnp.inf, jnp.float32)
        kx[rows:pad_rows, :] = inf_pad
        ky[rows:pad_rows, :] = inf_pad

    _sort_refs(kx, ix, True, pad_rows, chunk_rows, lgn, lgc)
    _sort_refs(ky, None, False, pad_rows, chunk_rows, lgn, lgc)
    _sort_refs(ix, ky, False, pad_rows, chunk_rows, lgn, lgc)

    o_ref[0] = ky[0:rows, :]


def _efdm_call(x3, y3, rows, pad_rows, chunk_rows, lgn, lgc, interpret=False):
    nrows = x3.shape[0]
    body = functools.partial(_efdm_row_kernel, rows=rows, pad_rows=pad_rows,
                             chunk_rows=chunk_rows, lgn=lgn, lgc=lgc)
    return pl.pallas_call(
        body,
        grid=(nrows,),
        in_specs=[
            pl.BlockSpec((1, rows, _LANES), lambda i: (i, 0, 0)),
            pl.BlockSpec((1, rows, _LANES), lambda i: (i, 0, 0)),
        ],
        out_specs=pl.BlockSpec((1, rows, _LANES), lambda i: (i, 0, 0)),
        out_shape=jax.ShapeDtypeStruct((nrows, rows, _LANES), jnp.float32),
        scratch_shapes=[
            pltpu.VMEM((pad_rows, _LANES), jnp.float32),
            pltpu.VMEM((pad_rows, _LANES), jnp.int32),
            pltpu.VMEM((pad_rows, _LANES), jnp.float32),
        ],
        compiler_params=pltpu.CompilerParams(
            dimension_semantics=("arbitrary",),
        ),
        interpret=interpret,
    )(x3, y3)


def kernel(x, y):
    B, C, W, H = x.shape
    n = W * H
    rows = n // _LANES
    lgn = (n - 1).bit_length()
    pad_rows = (1 << lgn) // _LANES
    chunk_rows = 64
    lgc = (chunk_rows * _LANES).bit_length() - 1

    x3 = x.reshape(B * C, rows, _LANES)
    y3 = y.reshape(B * C, rows, _LANES)
    out = _efdm_call(x3, y3, rows, pad_rows, chunk_rows, lgn, lgc)
    return out.reshape(B, C, W, H)


# speedup vs baseline: 1.3445x; 1.3445x over previous
import functools

import jax
import jax.numpy as jnp
from jax import lax
from jax.experimental import pallas as pl
from jax.experimental.pallas import tpu as pltpu

_LANES = 128


def _iota2(shape, offset=0):
    r = lax.broadcasted_iota(jnp.int32, shape, 0)
    c = lax.broadcasted_iota(jnp.int32, shape, 1)
    return r * _LANES + c + offset


def _bit_mask(shape, t):
    if t < 7:
        c = lax.broadcasted_iota(jnp.int32, shape, 1)
        return (c & (1 << t)) != 0
    r = lax.broadcasted_iota(jnp.int32, shape, 0)
    return (r & (1 << (t - 7))) != 0


def _take_small(bit, desc):
    if desc is None:
        return ~bit
    if isinstance(desc, bool):
        return bit if desc else ~bit
    return bit == desc


def _partner(a, t, bit):
    d = 1 << t
    if t < 7:
        ax, n, sh = 1, _LANES, d
    else:
        ax, n, sh = 0, a.shape[0], d >> 7
    return jnp.where(bit, pltpu.roll(a, sh, ax), pltpu.roll(a, n - sh, ax))


def _ce(k, v, t, desc, tie):
    bit = _bit_mask(k.shape, t)
    ts = _take_small(bit, desc)
    pk = _partner(k, t, bit)
    if v is None:
        mn = jnp.minimum(k, pk)
        mx = jnp.maximum(k, pk)
        return jnp.where(ts, mn, mx), None
    pv = _partner(v, t, bit)
    if tie:
        less = (k < pk) | ((k == pk) & (v < pv))
    else:
        less = k < pk
    keep = less == ts
    return jnp.where(keep, k, pk), jnp.where(keep, v, pv)


def _local_stage(k, v, s, desc, tie, lgc):
    for t in range(min(s - 1, lgc - 1), -1, -1):
        k, v = _ce(k, v, t, desc, tie)
    return k, v


def _sort_refs(k_ref, v_ref, tie, pad_rows, chunk_rows, lgn, lgc):
    nch = pad_rows // chunk_rows
    shape = (chunk_rows, _LANES)
    has_v = v_ref is not None

    def load(c_sl):
        return (k_ref[c_sl, :], v_ref[c_sl, :] if has_v else None)

    def store(c_sl, k, v):
        k_ref[c_sl, :] = k
        if has_v:
            v_ref[c_sl, :] = v

    def chunk_body(c, _):
        sl = pl.ds(c * chunk_rows, chunk_rows)
        k, v = load(sl)
        for s in range(1, lgc + 1):
            if s < lgc:
                desc = _bit_mask(shape, s)
            else:
                desc = (c & 1) == 1
            k, v = _local_stage(k, v, s, desc, tie, lgc)
        store(sl, k, v)
        return 0

    lax.fori_loop(0, nch, chunk_body, 0, unroll=False)

    for s in range(lgc + 1, lgn + 1):
        for t in range(s - 1, lgc - 1, -1):
            dch = 1 << (t - lgc)
            for c in range(nch):
                if c & dch:
                    continue
                p = c + dch
                desc = ((c >> (s - lgc)) & 1) == 1
                sa = slice(c * chunk_rows, (c + 1) * chunk_rows)
                sb = slice(p * chunk_rows, (p + 1) * chunk_rows)
                ka, va = load(sa)
                kb, vb = load(sb)
                if tie:
                    less = (ka < kb) | ((ka == kb) & (va < vb))
                else:
                    less = ka < kb
                if desc:
                    less = ~less
                store(sa, jnp.where(less, ka, kb),
                      jnp.where(less, va, vb) if has_v else None)
                store(sb, jnp.where(less, kb, ka),
                      jnp.where(less, vb, va) if has_v else None)

        def chunk_body2(c, _, s=s):
            sl = pl.ds(c * chunk_rows, chunk_rows)
            k, v = load(sl)
            desc = ((c >> (s - lgc)) & 1) == 1
            k, v = _local_stage(k, v, s, desc, tie, lgc)
            store(sl, k, v)
            return 0

        lax.fori_loop(0, nch, chunk_body2, 0, unroll=False)


def _efdm_row_kernel(x_ref, y_ref, o_ref, kx, ix, ky, *, rows, pad_rows,
                     chunk_rows, lgn, lgc):
    pad = pad_rows - rows
    kx[0:rows, :] = x_ref[0]
    ix[...] = _iota2((pad_rows, _LANES))
    ky[0:rows, :] = y_ref[0]
    if pad:
        inf_pad = jnp.full((pad, _LANES), jnp.inf, jnp.float32)
        kx[rows:pad_rows, :] = inf_pad
        ky[rows:pad_rows, :] = inf_pad

    _sort_refs(kx, ix, True, pad_rows, chunk_rows, lgn, lgc)
    _sort_refs(ky, None, False, pad_rows, chunk_rows, lgn, lgc)
    _sort_refs(ix, ky, False, pad_rows, chunk_rows, lgn, lgc)

    o_ref[0] = ky[0:rows, :]


def _efdm_call(x3, y3, rows, pad_rows, chunk_rows, lgn, lgc, interpret=False):
    nrows = x3.shape[0]
    body = functools.partial(_efdm_row_kernel, rows=rows, pad_rows=pad_rows,
                             chunk_rows=chunk_rows, lgn=lgn, lgc=lgc)
    return pl.pallas_call(
        body,
        grid=(nrows,),
        in_specs=[
            pl.BlockSpec((1, rows, _LANES), lambda i: (i, 0, 0)),
            pl.BlockSpec((1, rows, _LANES), lambda i: (i, 0, 0)),
        ],
        out_specs=pl.BlockSpec((1, rows, _LANES), lambda i: (i, 0, 0)),
        out_shape=jax.ShapeDtypeStruct((nrows, rows, _LANES), jnp.float32),
        scratch_shapes=[
            pltpu.VMEM((pad_rows, _LANES), jnp.float32),
            pltpu.VMEM((pad_rows, _LANES), jnp.int32),
            pltpu.VMEM((pad_rows, _LANES), jnp.float32),
        ],
        compiler_params=pltpu.CompilerParams(
            dimension_semantics=("arbitrary",),
        ),
        interpret=interpret,
    )(x3, y3)


def kernel(x, y):
    B, C, W, H = x.shape
    n = W * H
    rows = n // _LANES
    lgn = (n - 1).bit_length()
    pad_rows = (1 << lgn) // _LANES
    chunk_rows = 64
    lgc = (chunk_rows * _LANES).bit_length() - 1

    x3 = x.reshape(B * C, rows, _LANES)
    y3 = y.reshape(B * C, rows, _LANES)
    out = _efdm_call(x3, y3, rows, pad_rows, chunk_rows, lgn, lgc)
    return out.reshape(B, C, W, H)


# speedup vs baseline: 2.4397x; 1.8146x over previous
import functools

import jax
import jax.numpy as jnp
from jax import lax
from jax.experimental import pallas as pl
from jax.experimental.pallas import tpu as pltpu

_LANES = 128


def _iota2(shape, offset=0):
    r = lax.broadcasted_iota(jnp.int32, shape, 0)
    c = lax.broadcasted_iota(jnp.int32, shape, 1)
    return r * _LANES + c + offset


def _bit_mask(shape, t):
    if t < 7:
        c = lax.broadcasted_iota(jnp.int32, shape, 1)
        return (c & (1 << t)) != 0
    r = lax.broadcasted_iota(jnp.int32, shape, 0)
    return (r & (1 << (t - 7))) != 0


def _take_small(bit, desc):
    if desc is None:
        return ~bit
    if isinstance(desc, bool):
        return bit if desc else ~bit
    return bit == desc


def _partner(a, t, bit):
    d = 1 << t
    if t < 7:
        ax, n, sh = 1, _LANES, d
    else:
        ax, n, sh = 0, a.shape[0], d >> 7
    return jnp.where(bit, pltpu.roll(a, sh, ax), pltpu.roll(a, n - sh, ax))


def _ce(k, v, t, desc, tie):
    bit = _bit_mask(k.shape, t)
    ts = _take_small(bit, desc)
    pk = _partner(k, t, bit)
    if v is None:
        mn = jnp.minimum(k, pk)
        mx = jnp.maximum(k, pk)
        return jnp.where(ts, mn, mx), None
    pv = _partner(v, t, bit)
    if tie:
        less = (k < pk) | ((k == pk) & (v < pv))
    else:
        less = k < pk
    keep = less == ts
    return jnp.where(keep, k, pk), jnp.where(keep, v, pv)


def _local_stage(k, v, s, desc, tie, lgc):
    for t in range(min(s - 1, lgc - 1), -1, -1):
        k, v = _ce(k, v, t, desc, tie)
    return k, v


def _sort_refs(k_ref, v_ref, tie, pad_rows, chunk_rows, lgn, lgc):
    nch = pad_rows // chunk_rows
    shape = (chunk_rows, _LANES)
    has_v = v_ref is not None

    def load(c_sl):
        return (k_ref[c_sl, :], v_ref[c_sl, :] if has_v else None)

    def store(c_sl, k, v):
        k_ref[c_sl, :] = k
        if has_v:
            v_ref[c_sl, :] = v

    def chunk_body(c, _):
        sl = pl.ds(c * chunk_rows, chunk_rows)
        k, v = load(sl)
        for s in range(1, lgc + 1):
            if s < lgc:
                desc = _bit_mask(shape, s)
            else:
                desc = (c & 1) == 1
            k, v = _local_stage(k, v, s, desc, tie, lgc)
        store(sl, k, v)
        return 0

    lax.fori_loop(0, nch, chunk_body, 0, unroll=2)

    for s in range(lgc + 1, lgn + 1):
        for t in range(s - 1, lgc - 1, -1):
            dch = 1 << (t - lgc)
            for c in range(nch):
                if c & dch:
                    continue
                p = c + dch
                desc = ((c >> (s - lgc)) & 1) == 1
                sa = slice(c * chunk_rows, (c + 1) * chunk_rows)
                sb = slice(p * chunk_rows, (p + 1) * chunk_rows)
                ka, va = load(sa)
                kb, vb = load(sb)
                if tie:
                    less = (ka < kb) | ((ka == kb) & (va < vb))
                else:
                    less = ka < kb
                if desc:
                    less = ~less
                store(sa, jnp.where(less, ka, kb),
                      jnp.where(less, va, vb) if has_v else None)
                store(sb, jnp.where(less, kb, ka),
                      jnp.where(less, vb, va) if has_v else None)

        def chunk_body2(c, _, s=s):
            sl = pl.ds(c * chunk_rows, chunk_rows)
            k, v = load(sl)
            desc = ((c >> (s - lgc)) & 1) == 1
            k, v = _local_stage(k, v, s, desc, tie, lgc)
            store(sl, k, v)
            return 0

        lax.fori_loop(0, nch, chunk_body2, 0, unroll=2)


def _efdm_row_kernel(x_ref, y_ref, o_ref, kx, ix, ky, *, rows, pad_rows,
                     chunk_rows, lgn, lgc):
    pad = pad_rows - rows
    kx[0:rows, :] = x_ref[0]
    ix[...] = _iota2((pad_rows, _LANES))
    ky[0:rows, :] = y_ref[0]
    if pad:
        inf_pad = jnp.full((pad, _LANES), jnp.inf, jnp.float32)
        kx[rows:pad_rows, :] = inf_pad
        ky[rows:pad_rows, :] = inf_pad

    _sort_refs(kx, ix, True, pad_rows, chunk_rows, lgn, lgc)
    _sort_refs(ky, None, False, pad_rows, chunk_rows, lgn, lgc)
    _sort_refs(ix, ky, False, pad_rows, chunk_rows, lgn, lgc)

    o_ref[0] = ky[0:rows, :]


def _efdm_call(x3, y3, rows, pad_rows, chunk_rows, lgn, lgc, interpret=False):
    nrows = x3.shape[0]
    body = functools.partial(_efdm_row_kernel, rows=rows, pad_rows=pad_rows,
                             chunk_rows=chunk_rows, lgn=lgn, lgc=lgc)
    return pl.pallas_call(
        body,
        grid=(nrows,),
        in_specs=[
            pl.BlockSpec((1, rows, _LANES), lambda i: (i, 0, 0)),
            pl.BlockSpec((1, rows, _LANES), lambda i: (i, 0, 0)),
        ],
        out_specs=pl.BlockSpec((1, rows, _LANES), lambda i: (i, 0, 0)),
        out_shape=jax.ShapeDtypeStruct((nrows, rows, _LANES), jnp.float32),
        scratch_shapes=[
            pltpu.VMEM((pad_rows, _LANES), jnp.float32),
            pltpu.VMEM((pad_rows, _LANES), jnp.int32),
            pltpu.VMEM((pad_rows, _LANES), jnp.float32),
        ],
        compiler_params=pltpu.CompilerParams(
            dimension_semantics=("arbitrary",),
        ),
        interpret=interpret,
    )(x3, y3)


def kernel(x, y):
    B, C, W, H = x.shape
    n = W * H
    rows = n // _LANES
    lgn = (n - 1).bit_length()
    pad_rows = (1 << lgn) // _LANES
    chunk_rows = 64
    lgc = (chunk_rows * _LANES).bit_length() - 1

    x3 = x.reshape(B * C, rows, _LANES)
    y3 = y.reshape(B * C, rows, _LANES)
    out = _efdm_call(x3, y3, rows, pad_rows, chunk_rows, lgn, lgc)
    return out.reshape(B, C, W, H)
